# Initial kernel scaffold; baseline (speedup 1.0000x reference)
import jax
import jax.numpy as jnp
from jax.experimental import pallas as pl


def kernel(user_emb, item_emb, edge_weight, edge_index):
    raise NotImplementedError("write your pallas kernel here")


# speedup vs baseline: 3.8034x; 3.8034x over previous
import functools

import jax
import jax.numpy as jnp
from jax import lax
from jax.experimental import pallas as pl
from jax.experimental.pallas import tpu as pltpu
from jax.experimental.pallas import tpu_sc as plsc

DIM = 32
HALF = DIM // 2
NS = 16
CHUNK = 128


@functools.lru_cache(maxsize=None)
def _make_layer(n_nodes: int, e_pad: int):
    ept = e_pad // NS
    n_chunks = ept // CHUNK
    rows_per_tile = n_nodes // NS
    n_wb = 5
    wb = rows_per_tile // n_wb
    assert rows_per_tile % (8 * n_wb) == 0

    mesh = plsc.VectorSubcoreMesh(core_axis_name="c", subcore_axis_name="s")
    f32 = jnp.float32

    def body(t0, t1, src, dst, w, o0, o1,
             acc, src_v, dst_v, w_v, rows_v, bounce):
        c = lax.axis_index("c")
        s = lax.axis_index("s")
        row0 = s * rows_per_tile
        eb0 = s * ept

        def run(table, out):
            def zero_row(i, _):
                bounce[i, :] = jnp.zeros((16,), f32)
                return 0
            lax.fori_loop(0, wb, zero_row, 0)
            for j in range(n_wb):
                pltpu.sync_copy(bounce, acc.at[pl.ds(row0 + j * wb, wb)])
            plsc.subcore_barrier()

            def chunk(j, _):
                eb = eb0 + j * CHUNK
                pltpu.sync_copy(src.at[pl.ds(eb, CHUNK)], src_v)
                pltpu.sync_copy(dst.at[pl.ds(eb, CHUNK)], dst_v)
                pltpu.sync_copy(w.at[pl.ds(eb, CHUNK)], w_v)
                pltpu.sync_copy(table.at[src_v], rows_v)
                for g in range(CHUNK // 16):
                    wv = w_v[pl.ds(g * 16, 16)]
                    for i in range(16):
                        e = g * 16 + i
                        rows_v[e, :] = rows_v[e, :] * wv[i]
                pltpu.sync_copy(rows_v, acc.at[dst_v], add=True)
                return 0
            lax.fori_loop(0, n_chunks, chunk, 0)
            plsc.subcore_barrier()

            for j in range(n_wb):
                r = row0 + j * wb
                pltpu.sync_copy(acc.at[pl.ds(r, wb)], bounce)
                pltpu.sync_copy(bounce, out.at[pl.ds(r, wb)])

        @pl.when(c == 0)
        def _():
            run(t0, o0)

        @pl.when(c == 1)
        def _():
            run(t1, o1)

    return pl.kernel(
        body,
        out_type=[jax.ShapeDtypeStruct((n_nodes, HALF), f32)] * 2,
        mesh=mesh,
        compiler_params=pltpu.CompilerParams(use_tc_tiling_on_sc=False),
        scratch_types=[
            pltpu.VMEM_SHARED((n_nodes, HALF), f32),
            pltpu.VMEM((CHUNK,), jnp.int32),
            pltpu.VMEM((CHUNK,), jnp.int32),
            pltpu.VMEM((CHUNK,), f32),
            pltpu.VMEM((CHUNK, HALF), f32),
            pltpu.VMEM((wb, HALF), f32),
        ],
    )


@functools.lru_cache(maxsize=None)
def _make_mean(n_rows: int, n_cols: int):
    blk = 256
    grid = n_cols // blk

    def body(a0, b0, c0, d0, a1, b1, c1, d1, m0, m1):
        m0[...] = (a0[...] + b0[...] + c0[...] + d0[...]) * 0.25
        m1[...] = (a1[...] + b1[...] + c1[...] + d1[...]) * 0.25

    spec = pl.BlockSpec((n_rows, blk), lambda i: (0, i))
    return pl.pallas_call(
        body,
        grid=(grid,),
        in_specs=[spec] * 8,
        out_specs=[spec] * 2,
        out_shape=[jax.ShapeDtypeStruct((n_rows, n_cols), jnp.float32)] * 2,
    )


def kernel(user_emb, item_emb, edge_weight, edge_index):
    nu, d = user_emb.shape
    ni = item_emb.shape[0]
    n = nu + ni
    e = edge_weight.shape[0]
    assert d == DIM

    step = NS * CHUNK
    e_pad = ((e + step - 1) // step) * step
    pad = e_pad - e
    src = edge_index[0].astype(jnp.int32)
    dst = edge_index[1].astype(jnp.int32)
    if pad:
        ar = jnp.arange(pad, dtype=jnp.int32) % n
        src = jnp.concatenate([src, ar])
        dst = jnp.concatenate([dst, ar])
        w = jnp.concatenate([edge_weight, jnp.zeros((pad,), jnp.float32)])
    else:
        w = edge_weight

    n_pad = ((n + 639) // 640) * 640
    all_emb = jnp.concatenate(
        [user_emb, item_emb,
         jnp.zeros((n_pad - n, DIM), jnp.float32)], axis=0)
    t0 = all_emb[:, :HALF]
    t1 = all_emb[:, HALF:]

    layer = _make_layer(n_pad, e_pad)
    halves = [(t0, t1)]
    for _ in range(3):
        h0, h1 = layer(halves[-1][0], halves[-1][1], src, dst, w)
        halves.append((h0, h1))

    n_cols = 2560
    n_rows = n_pad * HALF // n_cols
    flat = lambda x: x.reshape(n_rows, n_cols)
    m0, m1 = _make_mean(n_rows, n_cols)(
        flat(halves[0][0]), flat(halves[1][0]), flat(halves[2][0]), flat(halves[3][0]),
        flat(halves[0][1]), flat(halves[1][1]), flat(halves[2][1]), flat(halves[3][1]),
    )
    mean = jnp.concatenate(
        [m0.reshape(n_pad, HALF), m1.reshape(n_pad, HALF)], axis=1)
    return mean[:nu], mean[nu:n]
